# baseline (speedup 1.0000x reference)
import functools

import jax
import jax.numpy as jnp
from jax.experimental import pallas as pl
from jax.experimental.pallas import tpu as pltpu

MAX_SIZE_ = 2048
RAMP_ = 32.0


def _mask_mul_kernel(cv_ref, x_ref, o_ref, *, block_rows, s):
    cv = cv_ref[0]
    r = jax.lax.broadcasted_iota(jnp.int32, (block_rows, 1), 0)
    r = (r + pl.program_id(0) * block_rows) & (s - 1)
    c = jax.lax.broadcasted_iota(jnp.int32, (block_rows, s), 1)
    i = jnp.minimum(r, s - 1 - r)
    left = i + jnp.where(r < s // 2, 0, 1)
    odm = (i.astype(jnp.float32) - (s // 2 - 1) + cv * MAX_SIZE_) / RAMP_ + 1.0
    odm = jnp.clip(odm, 0.0, 1.0)
    cond = (c >= left) & (c <= s - 1 - i)
    mask = jnp.where(cond, odm, 1.0)
    o_ref[...] = x_ref[...] * mask


@jax.jit
def kernel(x, current_val):
    b, h, s, _ = x.shape
    rows = b * h * s
    block_rows = 1536
    x2 = x.reshape(rows, s)
    grid = (rows // block_rows,)
    out = pl.pallas_call(
        functools.partial(_mask_mul_kernel, block_rows=block_rows, s=s),
        grid=grid,
        in_specs=[
            pl.BlockSpec(memory_space=pltpu.SMEM),
            pl.BlockSpec((block_rows, s), lambda n: (n, 0)),
        ],
        out_specs=pl.BlockSpec((block_rows, s), lambda n: (n, 0)),
        out_shape=jax.ShapeDtypeStruct((rows, s), x.dtype),
        compiler_params=pltpu.CompilerParams(
            dimension_semantics=("parallel",),
        ),
    )(current_val, x2)
    return out.reshape(b, h, s, s)
